# baseline (speedup 1.0000x reference)
import functools

import jax
import jax.numpy as jnp
from jax import lax
from jax.experimental import pallas as pl
from jax.experimental.pallas import tpu as pltpu
from jax.experimental.pallas import tpu_sc as plsc

DIM = 64
LANES = 16
NUM_CORES = 2
NUM_SUBCORES = 16
NUM_WORKERS = NUM_CORES * NUM_SUBCORES
BB = 128
SUP = 8
SC_ROWS = 2
CHUNK = SC_ROWS * BB
NG = BB // LANES


def _emb_fused(ids_t, sids_t, word, seg, batch, seq):
  n_sup = seq // SUP
  nbb = batch // BB
  mesh = plsc.VectorSubcoreMesh(
      core_axis_name="c", subcore_axis_name="s",
      num_cores=NUM_CORES, num_subcores=NUM_SUBCORES)

  @functools.partial(
      pl.kernel,
      out_type=jax.ShapeDtypeStruct((seq, DIM // 8, nbb, 8, BB), jnp.float32),
      mesh=mesh,
      scratch_types=[
          pltpu.VMEM((2, SUP, BB), jnp.int32),
          pltpu.VMEM((2, SUP, BB), jnp.int32),
          pltpu.VMEM((2, CHUNK, DIM), jnp.float32),
          pltpu.VMEM((2, SC_ROWS, DIM // 8, 1, 8, BB), jnp.float32),
          pltpu.VMEM((2, DIM), jnp.float32),
          pltpu.SemaphoreType.DMA,
          pltpu.SemaphoreType.DMA,
          pltpu.SemaphoreType.DMA,
          pltpu.SemaphoreType.DMA,
          pltpu.SemaphoreType.DMA,
      ],
      compiler_params=pltpu.CompilerParams(
          use_tc_tiling_on_sc=False, needs_layout_passes=False),
  )
  def body(ids_hbm, sids_hbm, word_hbm, seg_hbm, out_hbm,
           idx_b, sid_b, rows_v, out_v, seg_v,
           gsem0, gsem1, osem0, osem1, isem):
    gsem = [gsem0, gsem1]
    osem = [osem0, osem1]
    wid = lax.axis_index("s") * NUM_CORES + lax.axis_index("c")
    b0 = wid * BB
    pltpu.sync_copy(seg_hbm, seg_v)
    iota = lax.iota(jnp.int32, LANES)
    iota_dim = iota * DIM
    zeros = jnp.zeros((LANES,), jnp.int32)

    def stage_ids(sup, slot, issue):
      srow0 = sup * SUP
      if issue:
        h1 = pltpu.async_copy(
            ids_hbm.at[pl.ds(srow0, SUP), pl.ds(b0, BB)], idx_b.at[slot], isem)
        h2 = pltpu.async_copy(
            sids_hbm.at[pl.ds(srow0, SUP), pl.ds(b0, BB)], sid_b.at[slot],
            isem)
        del h1, h2
      else:
        pltpu.sync_copy(ids_hbm.at[pl.ds(srow0, SUP), pl.ds(b0, BB)],
                        idx_b.at[slot])
        pltpu.sync_copy(sids_hbm.at[pl.ds(srow0, SUP), pl.ds(b0, BB)],
                        sid_b.at[slot])

    def wait_ids():
      pltpu.make_async_copy(
          ids_hbm.at[pl.ds(0, SUP), pl.ds(0, BB)], idx_b.at[0], isem).wait()
      pltpu.make_async_copy(
          ids_hbm.at[pl.ds(0, SUP), pl.ds(0, BB)], sid_b.at[0], isem).wait()

    def issue_gathers(islot, rr0, dslot):
      for ri in range(SC_ROWS):
        pltpu.async_copy(
            word_hbm.at[idx_b.at[islot, rr0 + ri]],
            rows_v.at[dslot, pl.ds(ri * BB, BB)], gsem[dslot])

    def wait_gathers(dslot):
      pltpu.make_async_copy(
          word_hbm.at[pl.ds(0, CHUNK)], rows_v.at[dslot], gsem[dslot]).wait()

    def issue_writeback(s, srow):
      pltpu.async_copy(
          out_v.at[s],
          out_hbm.at[pl.ds(srow, SC_ROWS), :, pl.ds(wid, 1), :, :], osem[s])

    def wait_writeback(s):
      pltpu.make_async_copy(
          out_hbm.at[pl.ds(0, SC_ROWS), :, pl.ds(0, 1), :, :], out_v.at[s],
          osem[s]).wait()

    def compute(s, ps, q):
      for ri in range(SC_ROWS):
        rr = q * SC_ROWS + ri
        tok0 = (s * CHUNK + ri * BB) * DIM

        def group_body(g, c2, ri=ri, rr=rr, tok0=tok0):
          go = g * LANES
          svv = sid_b[ps, rr, pl.ds(go, LANES)]
          segbase = svv * DIM
          tokbase = iota_dim + (tok0 + go * DIM)
          for d in range(DIM):
            rowvals = plsc.load_gather(rows_v, [zeros, zeros, tokbase + d])
            segvals = plsc.load_gather(seg_v, [zeros, segbase + d])
            out_v[s, ri, d // 8, 0, d % 8, pl.ds(go, LANES)] = (
                rowvals + segvals)
          return c2

        lax.fori_loop(0, NG, group_body, 0)

    stage_ids(0, 0, issue=False)
    stage_ids(1, 1, issue=True)
    issue_gathers(0, 0, 0)

    def sup_body(p, carry):
      ps = lax.rem(p, 2)
      for q in range(SUP // SC_ROWS):
        s = q % 2
        c_srow = p * SUP + q * SC_ROWS
        wait_gathers(s)
        if q < (SUP // SC_ROWS) - 1:
          issue_gathers(ps, (q + 1) * SC_ROWS, 1 - s)
        else:
          wait_ids()
          issue_gathers(1 - ps, 0, 1 - s)
        if q < 2:
          @pl.when(p > 0)
          def _():
            wait_writeback(s)
        else:
          wait_writeback(s)
        compute(s, ps, q)
        issue_writeback(s, c_srow)
        if q == (SUP // SC_ROWS) - 1:
          stage_ids(lax.rem(p + 2, n_sup), ps, issue=True)
      return carry

    lax.fori_loop(0, n_sup, sup_body, 0)

    wait_writeback(0)
    wait_writeback(1)
    wait_gathers(0)
    wait_ids()

  return body(ids_t, sids_t, word, seg)


def kernel(input_ids, segment_ids, word_embeddings, segments_embedding):
  b, s = input_ids.shape
  ids_t = jnp.transpose(input_ids)
  sids_t = jnp.transpose(segment_ids)
  out5 = _emb_fused(ids_t, sids_t, word_embeddings, segments_embedding, b, s)
  out = jnp.transpose(out5, (2, 4, 0, 1, 3)).reshape(b, s, DIM)
  return (out, None)


# speedup vs baseline: 2.1956x; 2.1956x over previous
import functools

import jax
import jax.numpy as jnp
from jax import lax
from jax.experimental import pallas as pl
from jax.experimental.pallas import tpu as pltpu
from jax.experimental.pallas import tpu_sc as plsc

DIM = 64
LANES = 16
NUM_CORES = 2
NUM_SUBCORES = 16
NUM_WORKERS = NUM_CORES * NUM_SUBCORES
BB = 128
SUP = 8
SC_ROWS = 2
CHUNK = SC_ROWS * BB
NG = BB // LANES


def _emb_fused(ids4, sids4, word, seg, batch, seq):
  n_sup = seq // SUP
  nbb = batch // BB
  mesh = plsc.VectorSubcoreMesh(
      core_axis_name="c", subcore_axis_name="s",
      num_cores=NUM_CORES, num_subcores=NUM_SUBCORES)

  @functools.partial(
      pl.kernel,
      out_type=jax.ShapeDtypeStruct((seq, DIM // 8, nbb, 8, BB), jnp.float32),
      mesh=mesh,
      scratch_types=[
          pltpu.VMEM((2, 1, 1, SUP, BB), jnp.int32),
          pltpu.VMEM((2, 1, 1, SUP, BB), jnp.int32),
          pltpu.VMEM((2, CHUNK, DIM), jnp.float32),
          pltpu.VMEM((2, SC_ROWS, DIM // 8, 1, 8, BB), jnp.float32),
          pltpu.VMEM((2, DIM), jnp.float32),
          pltpu.SemaphoreType.DMA,
          pltpu.SemaphoreType.DMA,
          pltpu.SemaphoreType.DMA,
          pltpu.SemaphoreType.DMA,
          pltpu.SemaphoreType.DMA,
      ],
      compiler_params=pltpu.CompilerParams(
          use_tc_tiling_on_sc=False, needs_layout_passes=False),
  )
  def body(ids_hbm, sids_hbm, word_hbm, seg_hbm, out_hbm,
           idx_b, sid_b, rows_v, out_v, seg_v,
           gsem0, gsem1, osem0, osem1, isem):
    gsem = [gsem0, gsem1]
    osem = [osem0, osem1]
    wid = lax.axis_index("s") * NUM_CORES + lax.axis_index("c")
    pltpu.sync_copy(seg_hbm, seg_v)
    iota = lax.iota(jnp.int32, LANES)
    iota_dim = iota * DIM
    zeros = jnp.zeros((LANES,), jnp.int32)
    perms = [(iota + k) % LANES for k in range(LANES)]
    dparts = [(p // 8) * (8 * BB) + (p % 8) * BB for p in perms]

    def stage_ids(sup, slot, issue):
      src_i = ids_hbm.at[pl.ds(sup, 1), pl.ds(wid, 1), :, :]
      src_s = sids_hbm.at[pl.ds(sup, 1), pl.ds(wid, 1), :, :]
      if issue:
        pltpu.async_copy(src_i, idx_b.at[slot], isem)
        pltpu.async_copy(src_s, sid_b.at[slot], isem)
      else:
        pltpu.sync_copy(src_i, idx_b.at[slot])
        pltpu.sync_copy(src_s, sid_b.at[slot])

    def wait_ids():
      dummy = ids_hbm.at[pl.ds(0, 1), pl.ds(0, 1), :, :]
      pltpu.make_async_copy(dummy, idx_b.at[0], isem).wait()
      pltpu.make_async_copy(dummy, sid_b.at[0], isem).wait()

    def issue_gathers(islot, rr0, dslot):
      for ri in range(SC_ROWS):
        pltpu.async_copy(
            word_hbm.at[idx_b.at[islot, 0, 0, rr0 + ri]],
            rows_v.at[dslot, pl.ds(ri * BB, BB)], gsem[dslot])

    def wait_gathers(dslot):
      pltpu.make_async_copy(
          word_hbm.at[pl.ds(0, CHUNK)], rows_v.at[dslot], gsem[dslot]).wait()

    def issue_writeback(s, srow):
      pltpu.async_copy(
          out_v.at[s],
          out_hbm.at[pl.ds(srow, SC_ROWS), :, pl.ds(wid, 1), :, :], osem[s])

    def wait_writeback(s):
      pltpu.make_async_copy(
          out_hbm.at[pl.ds(0, SC_ROWS), :, pl.ds(0, 1), :, :], out_v.at[s],
          osem[s]).wait()

    def compute(s, ps, q):
      def ri_body(ri, c1):
        rr = q * SC_ROWS + ri
        tok0 = (s * CHUNK + ri * BB) * DIM
        slab = (s * SC_ROWS + ri) * DIM * BB

        def group_body(g, c2):
          go = g * LANES
          svv = sid_b[ps, 0, 0, rr, pl.ds(go, LANES)]
          seg0 = svv * DIM
          src0 = iota_dim + (tok0 + go * DIM)
          dst0 = iota + (go + slab)
          for k in range(LANES):
            srck = src0 + perms[k]
            segk = seg0 + perms[k]
            dstk = dst0 + dparts[k]
            for j in range(DIM // LANES):
              rowvals = plsc.load_gather(
                  rows_v, [zeros, zeros, srck + j * LANES])
              segvals = plsc.load_gather(seg_v, [zeros, segk + j * LANES])
              plsc.store_scatter(
                  out_v,
                  [zeros, zeros, zeros, zeros, zeros,
                   dstk + 2 * j * (8 * BB)],
                  rowvals + segvals)
          return c2

        lax.fori_loop(0, NG, group_body, 0)
        return c1

      lax.fori_loop(0, SC_ROWS, ri_body, 0)

    stage_ids(0, 0, issue=False)
    stage_ids(1, 1, issue=True)
    issue_gathers(0, 0, 0)

    def sup_body(p, carry):
      ps = lax.rem(p, 2)
      for q in range(SUP // SC_ROWS):
        s = q % 2
        c_srow = p * SUP + q * SC_ROWS
        wait_gathers(s)
        if q < (SUP // SC_ROWS) - 1:
          issue_gathers(ps, (q + 1) * SC_ROWS, 1 - s)
        else:
          wait_ids()
          issue_gathers(1 - ps, 0, 1 - s)
        if q < 2:
          @pl.when(p > 0)
          def _():
            wait_writeback(s)
        else:
          wait_writeback(s)
        compute(s, ps, q)
        issue_writeback(s, c_srow)
        if q == (SUP // SC_ROWS) - 1:
          stage_ids(lax.rem(p + 2, n_sup), ps, issue=True)
      return carry

    lax.fori_loop(0, n_sup, sup_body, 0)

    wait_writeback(0)
    wait_writeback(1)
    wait_gathers(0)
    wait_ids()

  return body(ids4, sids4, word, seg)


def kernel(input_ids, segment_ids, word_embeddings, segments_embedding):
  b, s = input_ids.shape
  def blockify(x):
    return jnp.transpose(
        jnp.reshape(jnp.transpose(x), (s // SUP, SUP, b // BB, BB)),
        (0, 2, 1, 3))

  out5 = _emb_fused(blockify(input_ids), blockify(segment_ids),
                    word_embeddings, segments_embedding, b, s)
  out = jnp.transpose(out5, (2, 4, 0, 1, 3)).reshape(b, s, DIM)
  return (out, None)
